# baseline (speedup 1.0000x reference)
import functools

import jax
import jax.numpy as jnp
import numpy as np
from jax import lax
from jax.experimental import pallas as pl
from jax.experimental.pallas import tpu as pltpu
from jax.experimental.pallas import tpu_sc as plsc

N_ATOMS = 2048
N_BONDS = 2048
N_ANGLES = 4096

NS = 16
BP = N_BONDS // NS
AP = N_ANGLES // NS
BG = BP // 16
AG = AP // 16

ACC = 8192
CHUNK = ACC // NS
E_SLOT = N_ATOMS * 3
E_TILE = E_SLOT // CHUNK

FMAX = np.float32(3.4028235e38)
PI = np.float32(3.14159265358979)
ACOS_C = [1.5707963050, -0.2145988016, 0.0889789874, -0.0501743046,
          0.0308918810, -0.0170881256, 0.0066700901, -0.0012624911]


def _sqrt(y):
    i = plsc.bitcast(y, jnp.int32)
    i = jnp.int32(0x5F3759DF) - (i >> 1)
    r = plsc.bitcast(i, jnp.float32)
    for _ in range(3):
        r = r * (1.5 - 0.5 * y * r * r)
    return y * r


def _acos(c):
    xa = jnp.abs(c)
    p = jnp.float32(ACOS_C[7])
    for a in ACOS_C[6::-1]:
        p = p * xa + jnp.float32(a)
    t = _sqrt(1.0 - xa) * p
    return jnp.where(c < 0, PI - t, t)


def _sc_body(dist_hbm, vec_hbm, ba_hbm, bb_hbm, bk0_hbm, breq_hbm,
             a1_hbm, a2_hbm, a3_hbm, ak0_hbm, ath0_hbm, out_hbm,
             ba_v, bb_v, bk0_v, breq_v,
             a1_v, a2_v, a3_v, ak0_v, ath0_v,
             didx_v, dval_v, vidx_v, vval_v,
             acc_v, blk_v, spmem, sem):
    wid = lax.axis_index("s")

    pltpu.sync_copy(ba_hbm.at[pl.ds(wid * BP, BP)], ba_v)
    pltpu.sync_copy(bb_hbm.at[pl.ds(wid * BP, BP)], bb_v)
    pltpu.sync_copy(bk0_hbm.at[pl.ds(wid * BP, BP)], bk0_v)
    pltpu.sync_copy(breq_hbm.at[pl.ds(wid * BP, BP)], breq_v)
    pltpu.sync_copy(a1_hbm.at[pl.ds(wid * AP, AP)], a1_v)
    pltpu.sync_copy(a2_hbm.at[pl.ds(wid * AP, AP)], a2_v)
    pltpu.sync_copy(a3_hbm.at[pl.ds(wid * AP, AP)], a3_v)
    pltpu.sync_copy(ak0_hbm.at[pl.ds(wid * AP, AP)], ak0_v)
    pltpu.sync_copy(ath0_hbm.at[pl.ds(wid * AP, AP)], ath0_v)

    for j in range(BG):
        s = pl.ds(j * 16, 16)
        a = ba_v[s]
        b = bb_v[s]
        q = (a * N_ATOMS + b) * 3
        didx_v[0, s] = a * N_ATOMS + b
        vidx_v[0, s] = q
        vidx_v[1, s] = q + 1
        vidx_v[2, s] = q + 2
    for j in range(AG):
        row, off = j // 8, (j % 8) * 16
        s = pl.ds(j * 16, 16)
        so = pl.ds(off, 16)
        i1 = a1_v[s]
        i2 = a2_v[s]
        i3 = a3_v[s]
        p21 = i2 * N_ATOMS + i1
        p23 = i2 * N_ATOMS + i3
        didx_v[1 + row, so] = p21
        didx_v[3 + row, so] = p23
        r = p21 * 3
        vidx_v[3 + 2 * 0 + row, so] = r
        vidx_v[5 + row, so] = r + 1
        vidx_v[7 + row, so] = r + 2
        rr = p23 * 3
        vidx_v[9 + row, so] = rr
        vidx_v[11 + row, so] = rr + 1
        vidx_v[13 + row, so] = rr + 2

    copies = []
    for k in range(5):
        copies.append(pltpu.async_copy(
            dist_hbm.at[didx_v.at[k]], dval_v.at[k], sem))
    for k in range(15):
        copies.append(pltpu.async_copy(
            vec_hbm.at[vidx_v.at[k]], vval_v.at[k], sem))

    def _zero(i, _):
        acc_v[pl.ds(i * 16, 16)] = jnp.zeros((16,), jnp.float32)
        return _
    lax.fori_loop(0, ACC // 16, _zero, None)

    for c in copies:
        c.wait()

    evec = jnp.zeros((16,), jnp.float32)

    for j in range(BG):
        s = pl.ds(j * 16, 16)
        d = dval_v[0, s]
        k0 = bk0_v[s]
        x = d - breq_v[s]
        evec = evec + k0 * x * x
        f = 2.0 * k0 * x
        fx = f * vval_v[0, s]
        fy = f * vval_v[1, s]
        fz = f * vval_v[2, s]
        ia = ba_v[s] * 3
        ib = bb_v[s] * 3
        plsc.addupdate_scatter(acc_v, [ia], fx)
        plsc.addupdate_scatter(acc_v, [ia + 1], fy)
        plsc.addupdate_scatter(acc_v, [ia + 2], fz)
        plsc.addupdate_scatter(acc_v, [ib], -fx)
        plsc.addupdate_scatter(acc_v, [ib + 1], -fy)
        plsc.addupdate_scatter(acc_v, [ib + 2], -fz)

    for j in range(AG):
        row, off = j // 8, (j % 8) * 16
        s = pl.ds(j * 16, 16)
        so = pl.ds(off, 16)
        d21 = dval_v[1 + row, so]
        d23 = dval_v[3 + row, so]
        x21 = vval_v[3 + row, so]
        y21 = vval_v[5 + row, so]
        z21 = vval_v[7 + row, so]
        x23 = vval_v[9 + row, so]
        y23 = vval_v[11 + row, so]
        z23 = vval_v[13 + row, so]
        cos = x21 * x23 + y21 * y23 + z21 * z23
        cos = jnp.minimum(jnp.maximum(cos, -1.0), 1.0)
        theta = _acos(cos)
        k0 = ak0_v[s]
        dth = theta - ath0_v[s]
        evec = evec + k0 * dth * dth
        sin = _sqrt(1.0 - cos * cos)
        coef = (-2.0 * k0 * dth) / sin
        coef = jnp.where(coef != coef, jnp.float32(0.0), coef)
        coef = jnp.minimum(jnp.maximum(coef, -FMAX), FMAX)
        c21 = coef / d21
        c23 = coef / d23
        f0x = c21 * (cos * x21 - x23)
        f0y = c21 * (cos * y21 - y23)
        f0z = c21 * (cos * z21 - z23)
        f2x = c23 * (cos * x23 - x21)
        f2y = c23 * (cos * y23 - y21)
        f2z = c23 * (cos * z23 - z21)
        i1 = a1_v[s] * 3
        i2 = a2_v[s] * 3
        i3 = a3_v[s] * 3
        plsc.addupdate_scatter(acc_v, [i1], f0x)
        plsc.addupdate_scatter(acc_v, [i1 + 1], f0y)
        plsc.addupdate_scatter(acc_v, [i1 + 2], f0z)
        plsc.addupdate_scatter(acc_v, [i2], -(f0x + f2x))
        plsc.addupdate_scatter(acc_v, [i2 + 1], -(f0y + f2y))
        plsc.addupdate_scatter(acc_v, [i2 + 2], -(f0z + f2z))
        plsc.addupdate_scatter(acc_v, [i3], f2x)
        plsc.addupdate_scatter(acc_v, [i3 + 1], f2y)
        plsc.addupdate_scatter(acc_v, [i3 + 2], f2z)

    acc_v[pl.ds(E_SLOT, 16)] = evec

    for c in range(NS):
        pltpu.sync_copy(acc_v.at[pl.ds(c * CHUNK, CHUNK)],
                        spmem.at[c, wid])
    plsc.subcore_barrier()
    pltpu.sync_copy(spmem.at[wid], blk_v)

    def _sum(i, _):
        off = i * 16
        tot = blk_v[0, pl.ds(off, 16)]
        for t in range(1, NS):
            tot = tot + blk_v[t, pl.ds(off, 16)]
        blk_v[0, pl.ds(off, 16)] = tot
        return _
    lax.fori_loop(0, CHUNK // 16, _sum, None)

    @pl.when(wid == E_TILE)
    def _finish_energy():
        loc = pl.ds(E_SLOT - E_TILE * CHUNK, 16)
        ev = blk_v[0, loc]
        blk_v[0, loc] = jnp.broadcast_to(jnp.sum(ev), (16,))

    pltpu.sync_copy(blk_v.at[0], out_hbm.at[pl.ds(wid * CHUNK, CHUNK)])


@jax.jit
def kernel(dist_mat, vector_mat, bond_params, angle_params, bond_idx, angle_idx):
    mesh = plsc.VectorSubcoreMesh(
        core_axis_name="c", subcore_axis_name="s", num_cores=1)
    sc_fn = pl.kernel(
        _sc_body,
        out_type=jax.ShapeDtypeStruct((ACC,), jnp.float32),
        mesh=mesh,
        compiler_params=pltpu.CompilerParams(needs_layout_passes=False),
        scratch_types=[
            pltpu.VMEM((BP,), jnp.int32),
            pltpu.VMEM((BP,), jnp.int32),
            pltpu.VMEM((BP,), jnp.float32),
            pltpu.VMEM((BP,), jnp.float32),
            pltpu.VMEM((AP,), jnp.int32),
            pltpu.VMEM((AP,), jnp.int32),
            pltpu.VMEM((AP,), jnp.int32),
            pltpu.VMEM((AP,), jnp.float32),
            pltpu.VMEM((AP,), jnp.float32),
            pltpu.VMEM((5, 128), jnp.int32),
            pltpu.VMEM((5, 128), jnp.float32),
            pltpu.VMEM((15, 128), jnp.int32),
            pltpu.VMEM((15, 128), jnp.float32),
            pltpu.VMEM((ACC,), jnp.float32),
            pltpu.VMEM((NS, CHUNK), jnp.float32),
            pltpu.VMEM_SHARED((NS, NS, CHUNK), jnp.float32),
            pltpu.SemaphoreType.DMA,
        ],
    )
    f32 = jnp.float32
    i32 = jnp.int32
    out = sc_fn(
        dist_mat.reshape(-1),
        vector_mat.reshape(-1),
        bond_idx[:, 0].astype(i32), bond_idx[:, 1].astype(i32),
        bond_params[:, 0].astype(f32), bond_params[:, 1].astype(f32),
        angle_idx[:, 0].astype(i32), angle_idx[:, 1].astype(i32),
        angle_idx[:, 2].astype(i32),
        angle_params[:, 0].astype(f32), angle_params[:, 1].astype(f32),
    )
    energy = out[E_SLOT]
    forces = out[:N_ATOMS * 3].reshape(N_ATOMS, 3)
    return energy, forces


# speedup vs baseline: 117.8432x; 117.8432x over previous
import jax
import jax.numpy as jnp
import numpy as np
from jax import lax
from jax.experimental import pallas as pl
from jax.experimental.pallas import tpu as pltpu
from jax.experimental.pallas import tpu_sc as plsc

N_ATOMS = 2048
N_BONDS = 2048
N_ANGLES = 4096

NS = 16
BP = N_BONDS // NS
AP = N_ANGLES // NS
BG = BP // 16
AG = AP // 16

ACC = 8192
CHUNK = ACC // NS
E_SLOT = N_ATOMS * 3
E_TILE = E_SLOT // CHUNK

FMAX = np.float32(3.4028235e38)
PI = np.float32(3.14159265358979)
ACOS_C = [1.5707963050, -0.2145988016, 0.0889789874, -0.0501743046,
          0.0308918810, -0.0170881256, 0.0066700901, -0.0012624911]


def _sqrt(y):
    i = plsc.bitcast(y, jnp.int32)
    i = jnp.int32(0x5F3759DF) - (i >> 1)
    r = plsc.bitcast(i, jnp.float32)
    for _ in range(3):
        r = r * (1.5 - 0.5 * y * r * r)
    return y * r


def _acos(c):
    xa = jnp.abs(c)
    p = jnp.float32(ACOS_C[7])
    for a in ACOS_C[6::-1]:
        p = p * xa + jnp.float32(a)
    t = _sqrt(1.0 - xa) * p
    return jnp.where(c < 0, PI - t, t)


def _sc_body(dist_hbm, vx_hbm, vy_hbm, vz_hbm, ba_hbm, bb_hbm, bk0_hbm, breq_hbm,
             a1_hbm, a2_hbm, a3_hbm, ak0_hbm, ath0_hbm, out_hbm,
             ba_v, bb_v, bk0_v, breq_v,
             a1_v, a2_v, a3_v, ak0_v, ath0_v,
             didx_v, dvalb_v, dval21_v, dval23_v,
             vbx_v, vby_v, vbz_v,
             v21x_v, v21y_v, v21z_v,
             v23x_v, v23y_v, v23z_v,
             acc_v, blk_v, spmem, sem):
    wid = lax.axis_index("s")

    pltpu.sync_copy(ba_hbm.at[pl.ds(wid * BP, BP)], ba_v)
    pltpu.sync_copy(bb_hbm.at[pl.ds(wid * BP, BP)], bb_v)
    pltpu.sync_copy(bk0_hbm.at[pl.ds(wid * BP, BP)], bk0_v)
    pltpu.sync_copy(breq_hbm.at[pl.ds(wid * BP, BP)], breq_v)
    pltpu.sync_copy(a1_hbm.at[pl.ds(wid * AP, AP)], a1_v)
    pltpu.sync_copy(a2_hbm.at[pl.ds(wid * AP, AP)], a2_v)
    pltpu.sync_copy(a3_hbm.at[pl.ds(wid * AP, AP)], a3_v)
    pltpu.sync_copy(ak0_hbm.at[pl.ds(wid * AP, AP)], ak0_v)
    pltpu.sync_copy(ath0_hbm.at[pl.ds(wid * AP, AP)], ath0_v)

    for j in range(BG):
        s = pl.ds(j * 16, 16)
        didx_v[0, s] = ba_v[s] * N_ATOMS + bb_v[s]
    for j in range(AG):
        row, off = j // 8, (j % 8) * 16
        s = pl.ds(j * 16, 16)
        so = pl.ds(off, 16)
        i2 = a2_v[s] * N_ATOMS
        didx_v[1 + row, so] = i2 + a1_v[s]
        didx_v[3 + row, so] = i2 + a3_v[s]

    copies = []
    for tbl, bond_dst, a21_dst, a23_dst in (
        (dist_hbm, dvalb_v, dval21_v, dval23_v),
        (vx_hbm, vbx_v, v21x_v, v23x_v),
        (vy_hbm, vby_v, v21y_v, v23y_v),
        (vz_hbm, vbz_v, v21z_v, v23z_v),
    ):
        copies.append(pltpu.async_copy(tbl.at[didx_v.at[0]], bond_dst, sem))
        copies.append(pltpu.async_copy(
            tbl.at[didx_v.at[1]], a21_dst.at[pl.ds(0, 128)], sem))
        copies.append(pltpu.async_copy(
            tbl.at[didx_v.at[2]], a21_dst.at[pl.ds(128, 128)], sem))
        copies.append(pltpu.async_copy(
            tbl.at[didx_v.at[3]], a23_dst.at[pl.ds(0, 128)], sem))
        copies.append(pltpu.async_copy(
            tbl.at[didx_v.at[4]], a23_dst.at[pl.ds(128, 128)], sem))

    def _zero(i, _):
        acc_v[pl.ds(i * 16, 16)] = jnp.zeros((16,), jnp.float32)
        return _
    lax.fori_loop(0, ACC // 16, _zero, None)

    for c in copies:
        c.wait()

    evec = jnp.zeros((16,), jnp.float32)

    for j in range(BG):
        s = pl.ds(j * 16, 16)
        d = dvalb_v[s]
        k0 = bk0_v[s]
        x = d - breq_v[s]
        evec = evec + k0 * x * x
        f = 2.0 * k0 * x
        fx = f * vbx_v[s]
        fy = f * vby_v[s]
        fz = f * vbz_v[s]
        ia = ba_v[s] * 3
        ib = bb_v[s] * 3
        plsc.addupdate_scatter(acc_v, [ia], fx)
        plsc.addupdate_scatter(acc_v, [ia + 1], fy)
        plsc.addupdate_scatter(acc_v, [ia + 2], fz)
        plsc.addupdate_scatter(acc_v, [ib], -fx)
        plsc.addupdate_scatter(acc_v, [ib + 1], -fy)
        plsc.addupdate_scatter(acc_v, [ib + 2], -fz)

    for j in range(AG):
        s = pl.ds(j * 16, 16)
        d21 = dval21_v[s]
        d23 = dval23_v[s]
        x21 = v21x_v[s]
        y21 = v21y_v[s]
        z21 = v21z_v[s]
        x23 = v23x_v[s]
        y23 = v23y_v[s]
        z23 = v23z_v[s]
        cos = x21 * x23 + y21 * y23 + z21 * z23
        cos = jnp.minimum(jnp.maximum(cos, -1.0), 1.0)
        theta = _acos(cos)
        k0 = ak0_v[s]
        dth = theta - ath0_v[s]
        evec = evec + k0 * dth * dth
        sin = _sqrt(1.0 - cos * cos)
        coef = (-2.0 * k0 * dth) / sin
        coef = jnp.where(coef != coef, jnp.float32(0.0), coef)
        coef = jnp.minimum(jnp.maximum(coef, -FMAX), FMAX)
        c21 = coef / d21
        c23 = coef / d23
        f0x = c21 * (cos * x21 - x23)
        f0y = c21 * (cos * y21 - y23)
        f0z = c21 * (cos * z21 - z23)
        f2x = c23 * (cos * x23 - x21)
        f2y = c23 * (cos * y23 - y21)
        f2z = c23 * (cos * z23 - z21)
        i1 = a1_v[s] * 3
        i2 = a2_v[s] * 3
        i3 = a3_v[s] * 3
        plsc.addupdate_scatter(acc_v, [i1], f0x)
        plsc.addupdate_scatter(acc_v, [i1 + 1], f0y)
        plsc.addupdate_scatter(acc_v, [i1 + 2], f0z)
        plsc.addupdate_scatter(acc_v, [i2], -(f0x + f2x))
        plsc.addupdate_scatter(acc_v, [i2 + 1], -(f0y + f2y))
        plsc.addupdate_scatter(acc_v, [i2 + 2], -(f0z + f2z))
        plsc.addupdate_scatter(acc_v, [i3], f2x)
        plsc.addupdate_scatter(acc_v, [i3 + 1], f2y)
        plsc.addupdate_scatter(acc_v, [i3 + 2], f2z)

    acc_v[pl.ds(E_SLOT, 16)] = evec

    for c in range(NS):
        pltpu.sync_copy(acc_v.at[pl.ds(c * CHUNK, CHUNK)],
                        spmem.at[c, wid])
    plsc.subcore_barrier()
    pltpu.sync_copy(spmem.at[wid], blk_v)

    def _sum(i, _):
        off = i * 16
        tot = blk_v[0, pl.ds(off, 16)]
        for t in range(1, NS):
            tot = tot + blk_v[t, pl.ds(off, 16)]
        blk_v[0, pl.ds(off, 16)] = tot
        return _
    lax.fori_loop(0, CHUNK // 16, _sum, None)

    @pl.when(wid == E_TILE)
    def _finish_energy():
        loc = pl.ds(E_SLOT - E_TILE * CHUNK, 16)
        ev = blk_v[0, loc]
        blk_v[0, loc] = jnp.broadcast_to(jnp.sum(ev), (16,))

    pltpu.sync_copy(blk_v.at[0], out_hbm.at[pl.ds(wid * CHUNK, CHUNK)])


@jax.jit
def kernel(dist_mat, vector_mat, bond_params, angle_params, bond_idx, angle_idx):
    mesh = plsc.VectorSubcoreMesh(
        core_axis_name="c", subcore_axis_name="s", num_cores=1)
    sc_fn = pl.kernel(
        _sc_body,
        out_type=jax.ShapeDtypeStruct((ACC,), jnp.float32),
        mesh=mesh,
        compiler_params=pltpu.CompilerParams(needs_layout_passes=False),
        scratch_types=[
            pltpu.VMEM((BP,), jnp.int32),
            pltpu.VMEM((BP,), jnp.int32),
            pltpu.VMEM((BP,), jnp.float32),
            pltpu.VMEM((BP,), jnp.float32),
            pltpu.VMEM((AP,), jnp.int32),
            pltpu.VMEM((AP,), jnp.int32),
            pltpu.VMEM((AP,), jnp.int32),
            pltpu.VMEM((AP,), jnp.float32),
            pltpu.VMEM((AP,), jnp.float32),
            pltpu.VMEM((5, 128), jnp.int32),
            pltpu.VMEM((BP,), jnp.float32),
            pltpu.VMEM((AP,), jnp.float32),
            pltpu.VMEM((AP,), jnp.float32),
            pltpu.VMEM((BP,), jnp.float32),
            pltpu.VMEM((BP,), jnp.float32),
            pltpu.VMEM((BP,), jnp.float32),
            pltpu.VMEM((AP,), jnp.float32),
            pltpu.VMEM((AP,), jnp.float32),
            pltpu.VMEM((AP,), jnp.float32),
            pltpu.VMEM((AP,), jnp.float32),
            pltpu.VMEM((AP,), jnp.float32),
            pltpu.VMEM((AP,), jnp.float32),
            pltpu.VMEM((ACC,), jnp.float32),
            pltpu.VMEM((NS, CHUNK), jnp.float32),
            pltpu.VMEM_SHARED((NS, NS, CHUNK), jnp.float32),
            pltpu.SemaphoreType.DMA,
        ],
    )
    f32 = jnp.float32
    i32 = jnp.int32
    out = sc_fn(
        dist_mat.reshape(-1),
        vector_mat[:, :, 0].reshape(-1),
        vector_mat[:, :, 1].reshape(-1),
        vector_mat[:, :, 2].reshape(-1),
        bond_idx[:, 0].astype(i32), bond_idx[:, 1].astype(i32),
        bond_params[:, 0].astype(f32), bond_params[:, 1].astype(f32),
        angle_idx[:, 0].astype(i32), angle_idx[:, 1].astype(i32),
        angle_idx[:, 2].astype(i32),
        angle_params[:, 0].astype(f32), angle_params[:, 1].astype(f32),
    )
    energy = out[E_SLOT]
    forces = out[:N_ATOMS * 3].reshape(N_ATOMS, 3)
    return energy, forces


# speedup vs baseline: 238.4896x; 2.0238x over previous
import jax
import jax.numpy as jnp
import numpy as np
from jax import lax
from jax.experimental import pallas as pl
from jax.experimental.pallas import tpu as pltpu
from jax.experimental.pallas import tpu_sc as plsc

N_ATOMS = 2048
N_BONDS = 2048
N_ANGLES = 4096

NS = 16
BP = N_BONDS // NS
AP = N_ANGLES // NS
BG = BP // 16
AG = AP // 16

NROWS = (N_ATOMS * N_ATOMS) // 128

ACC = 8192
CHUNK = ACC // NS
E_SLOT = N_ATOMS * 3
E_TILE = E_SLOT // CHUNK

FMAX = np.float32(3.4028235e38)
PI = np.float32(3.14159265358979)
ACOS_C = [1.5707963050, -0.2145988016, 0.0889789874, -0.0501743046,
          0.0308918810, -0.0170881256, 0.0066700901, -0.0012624911]


def _sqrt(y):
    i = plsc.bitcast(y, jnp.int32)
    i = jnp.int32(0x5F3759DF) - (i >> 1)
    r = plsc.bitcast(i, jnp.float32)
    for _ in range(3):
        r = r * (1.5 - 0.5 * y * r * r)
    return y * r


def _acos(c):
    xa = jnp.abs(c)
    p = jnp.float32(ACOS_C[7])
    for a in ACOS_C[6::-1]:
        p = p * xa + jnp.float32(a)
    t = _sqrt(1.0 - xa) * p
    return jnp.where(c < 0, PI - t, t)


def _trow(a, b):
    return ((a >> 3) << 7) + ((b >> 7) << 3) + (a & 7)


def _sc_body(dist4_hbm, vec5_hbm, ba_hbm, bb_hbm, bk0_hbm, breq_hbm,
             a1_hbm, a2_hbm, a3_hbm, ak0_hbm, ath0_hbm, out_hbm,
             ba_v, bb_v, bk0_v, breq_v,
             a1_v, a2_v, a3_v, ak0_v, ath0_v,
             bidx_v, aidx_v, blane_v, lane1_v, lane3_v,
             rb0_v, rb1_v,
             dvalb_v, dval21_v, dval23_v,
             vbx_v, vby_v, vbz_v,
             v21x_v, v21y_v, v21z_v,
             v23x_v, v23y_v, v23z_v,
             acc_v, blk_v, spmem, sem, sem2):
    wid = lax.axis_index("s")
    dist_t = dist4_hbm.reshape(NROWS, 128)
    vec_t = vec5_hbm.reshape(3 * NROWS, 128)

    pltpu.sync_copy(ba_hbm.at[pl.ds(wid * BP, BP)], ba_v)
    pltpu.sync_copy(bb_hbm.at[pl.ds(wid * BP, BP)], bb_v)
    pltpu.sync_copy(bk0_hbm.at[pl.ds(wid * BP, BP)], bk0_v)
    pltpu.sync_copy(breq_hbm.at[pl.ds(wid * BP, BP)], breq_v)
    pltpu.sync_copy(a1_hbm.at[pl.ds(wid * AP, AP)], a1_v)
    pltpu.sync_copy(a2_hbm.at[pl.ds(wid * AP, AP)], a2_v)
    pltpu.sync_copy(a3_hbm.at[pl.ds(wid * AP, AP)], a3_v)
    pltpu.sync_copy(ak0_hbm.at[pl.ds(wid * AP, AP)], ak0_v)
    pltpu.sync_copy(ath0_hbm.at[pl.ds(wid * AP, AP)], ath0_v)

    for j in range(BG):
        s = pl.ds(j * 16, 16)
        a = ba_v[s]
        b = bb_v[s]
        t = _trow(a, b)
        bidx_v[0, s] = t
        bidx_v[1, s] = t
        bidx_v[2, s] = t + NROWS
        bidx_v[3, s] = t + 2 * NROWS
        blane_v[s] = b & 127
    for j in range(AG):
        s = pl.ds(j * 16, 16)
        i1 = a1_v[s]
        i2 = a2_v[s]
        i3 = a3_v[s]
        t21 = _trow(i2, i1)
        t23 = _trow(i2, i3)
        aidx_v[0, s] = t21
        aidx_v[1, s] = t23
        aidx_v[2, s] = t21
        aidx_v[3, s] = t23
        aidx_v[4, s] = t21 + NROWS
        aidx_v[5, s] = t23 + NROWS
        aidx_v[6, s] = t21 + 2 * NROWS
        aidx_v[7, s] = t23 + 2 * NROWS
        lane1_v[s] = i1 & 127
        lane3_v[s] = i3 & 127

    def _zero(i, _):
        acc_v[pl.ds(i * 16, 16)] = jnp.zeros((16,), jnp.float32)
        return _
    lax.fori_loop(0, ACC // 16, _zero, None)

    passes = [
        (dist_t, True, 0, dvalb_v, blane_v),
        (vec_t, True, 1, vbx_v, blane_v),
        (vec_t, True, 2, vby_v, blane_v),
        (vec_t, True, 3, vbz_v, blane_v),
        (dist_t, False, 0, dval21_v, lane1_v),
        (dist_t, False, 1, dval23_v, lane3_v),
        (vec_t, False, 2, v21x_v, lane1_v),
        (vec_t, False, 3, v23x_v, lane3_v),
        (vec_t, False, 4, v21y_v, lane1_v),
        (vec_t, False, 5, v23y_v, lane3_v),
        (vec_t, False, 6, v21z_v, lane1_v),
        (vec_t, False, 7, v23z_v, lane3_v),
    ]

    def _fire(i):
        tbl, is_bond, k, _, _ = passes[i]
        buf = rb0_v if i % 2 == 0 else rb1_v
        s = sem if i % 2 == 0 else sem2
        if is_bond:
            return [pltpu.async_copy(
                tbl.at[bidx_v.at[k]], buf.at[pl.ds(0, 128)], s)]
        return [
            pltpu.async_copy(tbl.at[aidx_v.at[k, pl.ds(0, 128)]],
                             buf.at[pl.ds(0, 128)], s),
            pltpu.async_copy(tbl.at[aidx_v.at[k, pl.ds(128, 128)]],
                             buf.at[pl.ds(128, 128)], s),
        ]

    iota = lax.iota(jnp.int32, 16)
    pend = _fire(0)
    for i in range(len(passes)):
        nxt = _fire(i + 1) if i + 1 < len(passes) else []
        for c in pend:
            c.wait()
        _, is_bond, _, dst, laner = passes[i]
        buf = rb0_v if i % 2 == 0 else rb1_v
        for g in range(BG if is_bond else AG):
            s = pl.ds(g * 16, 16)
            dst[s] = plsc.load_gather(buf, [iota + g * 16, laner[s]])
        pend = nxt

    evec = jnp.zeros((16,), jnp.float32)

    for j in range(BG):
        s = pl.ds(j * 16, 16)
        d = dvalb_v[s]
        k0 = bk0_v[s]
        x = d - breq_v[s]
        evec = evec + k0 * x * x
        f = 2.0 * k0 * x
        fx = f * vbx_v[s]
        fy = f * vby_v[s]
        fz = f * vbz_v[s]
        ia = ba_v[s] * 3
        ib = bb_v[s] * 3
        plsc.addupdate_scatter(acc_v, [ia], fx)
        plsc.addupdate_scatter(acc_v, [ia + 1], fy)
        plsc.addupdate_scatter(acc_v, [ia + 2], fz)
        plsc.addupdate_scatter(acc_v, [ib], -fx)
        plsc.addupdate_scatter(acc_v, [ib + 1], -fy)
        plsc.addupdate_scatter(acc_v, [ib + 2], -fz)

    for j in range(AG):
        s = pl.ds(j * 16, 16)
        d21 = dval21_v[s]
        d23 = dval23_v[s]
        x21 = v21x_v[s]
        y21 = v21y_v[s]
        z21 = v21z_v[s]
        x23 = v23x_v[s]
        y23 = v23y_v[s]
        z23 = v23z_v[s]
        cos = x21 * x23 + y21 * y23 + z21 * z23
        cos = jnp.minimum(jnp.maximum(cos, -1.0), 1.0)
        theta = _acos(cos)
        k0 = ak0_v[s]
        dth = theta - ath0_v[s]
        evec = evec + k0 * dth * dth
        sin = _sqrt(1.0 - cos * cos)
        coef = (-2.0 * k0 * dth) / sin
        coef = jnp.where(coef != coef, jnp.float32(0.0), coef)
        coef = jnp.minimum(jnp.maximum(coef, -FMAX), FMAX)
        c21 = coef / d21
        c23 = coef / d23
        f0x = c21 * (cos * x21 - x23)
        f0y = c21 * (cos * y21 - y23)
        f0z = c21 * (cos * z21 - z23)
        f2x = c23 * (cos * x23 - x21)
        f2y = c23 * (cos * y23 - y21)
        f2z = c23 * (cos * z23 - z21)
        i1 = a1_v[s] * 3
        i2 = a2_v[s] * 3
        i3 = a3_v[s] * 3
        plsc.addupdate_scatter(acc_v, [i1], f0x)
        plsc.addupdate_scatter(acc_v, [i1 + 1], f0y)
        plsc.addupdate_scatter(acc_v, [i1 + 2], f0z)
        plsc.addupdate_scatter(acc_v, [i2], -(f0x + f2x))
        plsc.addupdate_scatter(acc_v, [i2 + 1], -(f0y + f2y))
        plsc.addupdate_scatter(acc_v, [i2 + 2], -(f0z + f2z))
        plsc.addupdate_scatter(acc_v, [i3], f2x)
        plsc.addupdate_scatter(acc_v, [i3 + 1], f2y)
        plsc.addupdate_scatter(acc_v, [i3 + 2], f2z)

    acc_v[pl.ds(E_SLOT, 16)] = evec

    for c in range(NS):
        pltpu.sync_copy(acc_v.at[pl.ds(c * CHUNK, CHUNK)],
                        spmem.at[c, wid])
    plsc.subcore_barrier()
    pltpu.sync_copy(spmem.at[wid], blk_v)

    def _sum(i, _):
        off = i * 16
        tot = blk_v[0, pl.ds(off, 16)]
        for t in range(1, NS):
            tot = tot + blk_v[t, pl.ds(off, 16)]
        blk_v[0, pl.ds(off, 16)] = tot
        return _
    lax.fori_loop(0, CHUNK // 16, _sum, None)

    @pl.when(wid == E_TILE)
    def _finish_energy():
        loc = pl.ds(E_SLOT - E_TILE * CHUNK, 16)
        ev = blk_v[0, loc]
        blk_v[0, loc] = jnp.broadcast_to(jnp.sum(ev), (16,))

    pltpu.sync_copy(blk_v.at[0], out_hbm.at[pl.ds(wid * CHUNK, CHUNK)])


@jax.jit
def kernel(dist_mat, vector_mat, bond_params, angle_params, bond_idx, angle_idx):
    mesh = plsc.VectorSubcoreMesh(
        core_axis_name="c", subcore_axis_name="s", num_cores=1)
    sc_fn = pl.kernel(
        _sc_body,
        out_type=jax.ShapeDtypeStruct((ACC,), jnp.float32),
        mesh=mesh,
        compiler_params=pltpu.CompilerParams(needs_layout_passes=False),
        scratch_types=[
            pltpu.VMEM((BP,), jnp.int32),
            pltpu.VMEM((BP,), jnp.int32),
            pltpu.VMEM((BP,), jnp.float32),
            pltpu.VMEM((BP,), jnp.float32),
            pltpu.VMEM((AP,), jnp.int32),
            pltpu.VMEM((AP,), jnp.int32),
            pltpu.VMEM((AP,), jnp.int32),
            pltpu.VMEM((AP,), jnp.float32),
            pltpu.VMEM((AP,), jnp.float32),
            pltpu.VMEM((4, BP), jnp.int32),
            pltpu.VMEM((8, AP), jnp.int32),
            pltpu.VMEM((BP,), jnp.int32),
            pltpu.VMEM((AP,), jnp.int32),
            pltpu.VMEM((AP,), jnp.int32),
            pltpu.VMEM((AP, 128), jnp.float32),
            pltpu.VMEM((AP, 128), jnp.float32),
            pltpu.VMEM((BP,), jnp.float32),
            pltpu.VMEM((AP,), jnp.float32),
            pltpu.VMEM((AP,), jnp.float32),
            pltpu.VMEM((BP,), jnp.float32),
            pltpu.VMEM((BP,), jnp.float32),
            pltpu.VMEM((BP,), jnp.float32),
            pltpu.VMEM((AP,), jnp.float32),
            pltpu.VMEM((AP,), jnp.float32),
            pltpu.VMEM((AP,), jnp.float32),
            pltpu.VMEM((AP,), jnp.float32),
            pltpu.VMEM((AP,), jnp.float32),
            pltpu.VMEM((AP,), jnp.float32),
            pltpu.VMEM((ACC,), jnp.float32),
            pltpu.VMEM((NS, CHUNK), jnp.float32),
            pltpu.VMEM_SHARED((NS, NS, CHUNK), jnp.float32),
            pltpu.SemaphoreType.DMA,
            pltpu.SemaphoreType.DMA,
        ],
    )
    f32 = jnp.float32
    i32 = jnp.int32
    dist4 = dist_mat.reshape(N_ATOMS // 8, 8, N_ATOMS // 128, 128
                             ).transpose(0, 2, 1, 3)
    vec5 = vector_mat.transpose(2, 0, 1).reshape(
        3, N_ATOMS // 8, 8, N_ATOMS // 128, 128).transpose(0, 1, 3, 2, 4)
    out = sc_fn(
        dist4,
        vec5,
        bond_idx[:, 0].astype(i32), bond_idx[:, 1].astype(i32),
        bond_params[:, 0].astype(f32), bond_params[:, 1].astype(f32),
        angle_idx[:, 0].astype(i32), angle_idx[:, 1].astype(i32),
        angle_idx[:, 2].astype(i32),
        angle_params[:, 0].astype(f32), angle_params[:, 1].astype(f32),
    )
    energy = out[E_SLOT]
    forces = out[:N_ATOMS * 3].reshape(N_ATOMS, 3)
    return energy, forces
